# baseline (speedup 1.0000x reference)
import functools

import jax
import jax.numpy as jnp
import numpy as np
from jax import lax
from jax.experimental import pallas as pl
from jax.experimental.pallas import tpu as pltpu
from jax.experimental.pallas import tpu_sc as plsc

_R = 32
_C = 1000000
_BLK = 8192
_W = 128
_NCH = _BLK // _W
_NBF = _C // _BLK
_TAIL_START = _NBF * _BLK
_TAIL_BLK = 1024
_TAIL_NCH = _TAIL_BLK // _W

_B = 38
_NSC = _B * _BLK
_NB1 = _NBF - _B
_S0 = _NB1 * _BLK
_CH = 4096
_U = 4

_K1 = np.int32(42)
_K2 = np.int32(np.uint32(42) ^ np.uint32(0x1BD11BDA))
_ROT0 = (13, 15, 26, 6)
_ROT1 = (17, 29, 16, 24)
_TINY = np.float32(np.finfo(np.float32).tiny)
_NEG_INF = np.float32(-np.inf)
_IMAX = np.int32(2**31 - 1)


def _rotl(v, r):
    return lax.shift_left(v, np.int32(r)) | lax.shift_right_logical(
        v, np.int32(32 - r))


def _rounds(x0, x1, rots):
    for r in rots:
        x0 = x0 + x1
        x1 = _rotl(x1, r)
        x1 = x0 ^ x1
    return x0, x1


def _threefry_bits(x1):
    x0 = x1
    x1 = x0 ^ _rotl(x1, 13)
    for r in _ROT0[1:]:
        x0 = x0 + x1
        x1 = _rotl(x1, r)
        x1 = x0 ^ x1
    x0 = x0 + _K1
    x1 = x1 + np.int32(_K2 + np.uint32(1))
    x0, x1 = _rounds(x0, x1, _ROT1)
    x0 = x0 + _K2
    x1 = x1 + np.int32(2)
    x0, x1 = _rounds(x0, x1, _ROT0)
    x1 = x1 + np.int32(_K1 + np.uint32(3))
    x0, x1 = _rounds(x0, x1, _ROT1)
    x0 = x0 + _K1
    x1 = x1 + np.int32(_K2 + np.uint32(4))
    x0, x1 = _rounds(x0, x1, _ROT0)
    x0 = x0 + _K2
    x1 = x1 + np.int32(5)
    return x0 ^ x1


def _bits_to_score(bits, xv):
    fb = lax.shift_right_logical(bits, np.int32(9)) | np.int32(0x3F800000)
    f = lax.bitcast_convert_type(fb, jnp.float32) - np.float32(1.0)
    u = f + _TINY
    return -jnp.log(-jnp.log(u)) + jnp.log(xv)


def _score(xv, x1_0):
    return _bits_to_score(_threefry_bits(x1_0), xv)


def _sc_bits_body(o_hbm, va_ref, vb_ref, sa, sb):
    w = lax.axis_index("s") * 2 + lax.axis_index("c")
    base = w * np.int32(_C) + np.int32(_S0 + 42)
    obase = w * np.int32(_NSC)
    lanes = lax.iota(jnp.int32, 16)

    def fill(v_ref, b0):
        def vec_body(j, c2):
            pos = j * np.int32(16 * _U)
            for k in range(_U):
                x1 = (b0 + pos + np.int32(16 * k)) + lanes
                v_ref[pl.ds(pos + np.int32(16 * k), 16)] = _threefry_bits(x1)
            return c2

        lax.fori_loop(0, _CH // (16 * _U), vec_body, 0)

    def pair_body(ci, carry):
        off = ci * np.int32(2 * _CH)

        @pl.when(ci > 0)
        def _wait_a():
            pltpu.make_async_copy(va_ref, o_hbm.at[pl.ds(obase, _CH)],
                                  sa).wait()

        fill(va_ref, base + off)
        pltpu.async_copy(va_ref, o_hbm.at[pl.ds(obase + off, _CH)], sa)

        @pl.when(ci > 0)
        def _wait_b():
            pltpu.make_async_copy(vb_ref, o_hbm.at[pl.ds(obase, _CH)],
                                  sb).wait()

        fill(vb_ref, base + off + np.int32(_CH))
        pltpu.async_copy(
            vb_ref, o_hbm.at[pl.ds(obase + off + np.int32(_CH), _CH)], sb)
        return carry

    lax.fori_loop(0, _NSC // (2 * _CH), pair_body, 0)
    pltpu.make_async_copy(va_ref, o_hbm.at[pl.ds(obase, _CH)], sa).wait()
    pltpu.make_async_copy(vb_ref, o_hbm.at[pl.ds(obase, _CH)], sb).wait()


def _sc_bits():
    mesh = plsc.VectorSubcoreMesh(core_axis_name="c", subcore_axis_name="s")
    return pl.kernel(
        _sc_bits_body,
        mesh=mesh,
        out_type=jax.ShapeDtypeStruct((_R * _NSC,), jnp.int32),
        scratch_types=[pltpu.VMEM((_CH,), jnp.int32),
                       pltpu.VMEM((_CH,), jnp.int32),
                       pltpu.SemaphoreType.DMA,
                       pltpu.SemaphoreType.DMA],
    )()


def _run_chunks(x_ref, basec, start, chunks):
    bv = bc = None
    for ch in chunks:
        xv = x_ref[:, ch * _W:(ch + 1) * _W]
        c = basec + (start + np.int32(ch * _W))
        s = _score(xv, c)
        if bv is None:
            bv, bc = s, c
        else:
            upd = s > bv
            bv = jnp.where(upd, s, bv)
            bc = jnp.where(upd, c, bc)
    return bv, bc


def _tc1_kernel(base_ref, x_ref, xt_ref, bvo_ref, bco_ref, bv_ref, bc_ref):
    p = pl.program_id(0)
    basec = base_ref[...]

    start = p * np.int32(_BLK)
    h = _NCH // 2
    bva, bca = _run_chunks(x_ref, basec, start, range(h))
    bvb, bcb = _run_chunks(x_ref, basec, start, range(h, _NCH))
    updh = bvb > bva
    bv = jnp.where(updh, bvb, bva)
    bc = jnp.where(updh, bcb, bca)

    @pl.when(p == 0)
    def _init():
        bv_ref[...] = bv
        bc_ref[...] = bc

    @pl.when(p > 0)
    def _merge():
        ov = bv_ref[...]
        upd = bv > ov
        bv_ref[...] = jnp.where(upd, bv, ov)
        bc_ref[...] = jnp.where(upd, bc, bc_ref[...])

    @pl.when(p == _NB1 - 1)
    def _fin():
        mv = bv_ref[...]
        mc = bc_ref[...]
        lane = basec - basec[:, :1]
        for tc in range(_TAIL_NCH):
            col0 = _TAIL_START + tc * _W
            xv = xt_ref[:, tc * _W:(tc + 1) * _W]
            c = basec + np.int32(col0)
            s = _score(xv, c)
            s = jnp.where(lane < np.int32(_C - col0), s, _NEG_INF)
            upd = s > mv
            mv = jnp.where(upd, s, mv)
            mc = jnp.where(upd, c, mc)
        bvo_ref[...] = mv
        bco_ref[...] = mc


def _tc2_kernel(base_ref, bvi_ref, bci_ref, x_ref, bits_ref, o_ref,
                bv_ref, bc_ref):
    p = pl.program_id(0)
    basec = base_ref[...]

    @pl.when(p == 0)
    def _init():
        bv_ref[...] = bvi_ref[...]
        bc_ref[...] = bci_ref[...]

    mv = bv_ref[...]
    mc = bc_ref[...]
    start = np.int32(_S0) + p * np.int32(_BLK)
    for ch in range(_NCH):
        bits = bits_ref[:, ch * _W:(ch + 1) * _W]
        xv = x_ref[:, ch * _W:(ch + 1) * _W]
        c = basec + (start + np.int32(ch * _W))
        s = _bits_to_score(bits, xv)
        upd = (s > mv) | ((s == mv) & (c < mc))
        mv = jnp.where(upd, s, mv)
        mc = jnp.where(upd, c, mc)
    bv_ref[...] = mv
    bc_ref[...] = mc

    @pl.when(p == _B - 1)
    def _fin():
        m = jnp.max(mv, axis=1, keepdims=True)
        arg = jnp.min(jnp.where(mv == m, mc, _IMAX), axis=1, keepdims=True)
        o_ref[...] = arg - basec[:, :1]


@jax.jit
def kernel(x):
    base = (np.arange(_R, dtype=np.int32)[:, None] * _C +
            np.arange(_W, dtype=np.int32)[None, :] + 42)
    basec = jnp.asarray(base)

    sc_bits = _sc_bits().reshape(_R, _NSC)

    bv1, bc1 = pl.pallas_call(
        _tc1_kernel,
        grid=(_NB1,),
        in_specs=[
            pl.BlockSpec((_R, _W), lambda p: (0, 0)),
            pl.BlockSpec((_R, _BLK), lambda p: (0, p)),
            pl.BlockSpec((_R, _TAIL_BLK),
                         lambda p: (0, _TAIL_START // _TAIL_BLK)),
        ],
        out_specs=[
            pl.BlockSpec((_R, _W), lambda p: (0, 0)),
            pl.BlockSpec((_R, _W), lambda p: (0, 0)),
        ],
        out_shape=[
            jax.ShapeDtypeStruct((_R, _W), jnp.float32),
            jax.ShapeDtypeStruct((_R, _W), jnp.int32),
        ],
        scratch_shapes=[
            pltpu.VMEM((_R, _W), jnp.float32),
            pltpu.VMEM((_R, _W), jnp.int32),
        ],
    )(basec, x, x)

    return pl.pallas_call(
        _tc2_kernel,
        grid=(_B,),
        in_specs=[
            pl.BlockSpec((_R, _W), lambda p: (0, 0)),
            pl.BlockSpec((_R, _W), lambda p: (0, 0)),
            pl.BlockSpec((_R, _W), lambda p: (0, 0)),
            pl.BlockSpec((_R, _BLK), lambda p: (0, _NB1 + p)),
            pl.BlockSpec((_R, _BLK), lambda p: (0, p)),
        ],
        out_specs=pl.BlockSpec((_R, 1), lambda p: (0, 0)),
        out_shape=jax.ShapeDtypeStruct((_R, 1), jnp.int32),
        scratch_shapes=[
            pltpu.VMEM((_R, _W), jnp.float32),
            pltpu.VMEM((_R, _W), jnp.int32),
        ],
    )(basec, bv1, bc1, x, sc_bits)


# speedup vs baseline: 1.0604x; 1.0604x over previous
import functools

import jax
import jax.numpy as jnp
import numpy as np
from jax import lax
from jax.experimental import pallas as pl
from jax.experimental.pallas import tpu as pltpu
from jax.experimental.pallas import tpu_sc as plsc

_R = 32
_C = 1000000
_BLK = 8192
_W = 128
_NCH = _BLK // _W
_NBF = _C // _BLK
_TAIL_START = _NBF * _BLK
_TAIL_BLK = 1024
_TAIL_NCH = _TAIL_BLK // _W

_B = 38
_NSC = _B * _BLK
_NB1 = _NBF - _B
_S0 = _NB1 * _BLK
_CH = 4096
_U = 8

_K1 = np.int32(42)
_K2 = np.int32(np.uint32(42) ^ np.uint32(0x1BD11BDA))
_ROT0 = (13, 15, 26, 6)
_ROT1 = (17, 29, 16, 24)
_TINY = np.float32(np.finfo(np.float32).tiny)
_NEG_INF = np.float32(-np.inf)
_IMAX = np.int32(2**31 - 1)


def _rotl(v, r):
    return lax.shift_left(v, np.int32(r)) | lax.shift_right_logical(
        v, np.int32(32 - r))


def _rounds(x0, x1, rots):
    for r in rots:
        x0 = x0 + x1
        x1 = _rotl(x1, r)
        x1 = x0 ^ x1
    return x0, x1


def _threefry_bits(x1):
    x0 = x1
    x1 = x0 ^ _rotl(x1, 13)
    for r in _ROT0[1:]:
        x0 = x0 + x1
        x1 = _rotl(x1, r)
        x1 = x0 ^ x1
    x0 = x0 + _K1
    x1 = x1 + np.int32(_K2 + np.uint32(1))
    x0, x1 = _rounds(x0, x1, _ROT1)
    x0 = x0 + _K2
    x1 = x1 + np.int32(2)
    x0, x1 = _rounds(x0, x1, _ROT0)
    x1 = x1 + np.int32(_K1 + np.uint32(3))
    x0, x1 = _rounds(x0, x1, _ROT1)
    x0 = x0 + _K1
    x1 = x1 + np.int32(_K2 + np.uint32(4))
    x0, x1 = _rounds(x0, x1, _ROT0)
    x0 = x0 + _K2
    x1 = x1 + np.int32(5)
    return x0 ^ x1


def _bits_to_score(bits, xv):
    fb = lax.shift_right_logical(bits, np.int32(9)) | np.int32(0x3F800000)
    f = lax.bitcast_convert_type(fb, jnp.float32) - np.float32(1.0)
    u = f + _TINY
    return -jnp.log(-jnp.log(u)) + jnp.log(xv)


def _score(xv, x1_0):
    return _bits_to_score(_threefry_bits(x1_0), xv)


def _sc_bits_body(o_hbm, va_ref, vb_ref, sa, sb):
    w = lax.axis_index("s") * 2 + lax.axis_index("c")
    base = w * np.int32(_C) + np.int32(_S0 + 42)
    lanes = lax.iota(jnp.int32, 16)

    def fill(v_ref, b0):
        def vec_body(j, c2):
            pos = j * np.int32(16 * _U)
            for k in range(_U):
                x1 = (b0 + pos + np.int32(16 * k)) + lanes
                v_ref[pl.ds(pos + np.int32(16 * k), 16)] = _threefry_bits(x1)
            return c2

        lax.fori_loop(0, _CH // (16 * _U), vec_body, 0)

    def pair_body(ci, carry):
        off = ci * np.int32(2 * _CH)

        @pl.when(ci > 0)
        def _wait_a():
            pltpu.make_async_copy(va_ref, o_hbm.at[w, pl.ds(0, _CH)],
                                  sa).wait()

        fill(va_ref, base + off)
        pltpu.async_copy(va_ref, o_hbm.at[w, pl.ds(off, _CH)], sa)

        @pl.when(ci > 0)
        def _wait_b():
            pltpu.make_async_copy(vb_ref, o_hbm.at[w, pl.ds(0, _CH)],
                                  sb).wait()

        fill(vb_ref, base + off + np.int32(_CH))
        pltpu.async_copy(
            vb_ref, o_hbm.at[w, pl.ds(off + np.int32(_CH), _CH)], sb)
        return carry

    lax.fori_loop(0, _NSC // (2 * _CH), pair_body, 0)
    pltpu.make_async_copy(va_ref, o_hbm.at[w, pl.ds(0, _CH)], sa).wait()
    pltpu.make_async_copy(vb_ref, o_hbm.at[w, pl.ds(0, _CH)], sb).wait()


def _sc_bits():
    mesh = plsc.VectorSubcoreMesh(core_axis_name="c", subcore_axis_name="s")
    return pl.kernel(
        _sc_bits_body,
        mesh=mesh,
        out_type=jax.ShapeDtypeStruct((_R, _NSC), jnp.int32),
        scratch_types=[pltpu.VMEM((_CH,), jnp.int32),
                       pltpu.VMEM((_CH,), jnp.int32),
                       pltpu.SemaphoreType.DMA,
                       pltpu.SemaphoreType.DMA],
    )()


def _run_chunks(x_ref, basec, start, chunks):
    bv = bc = None
    for ch in chunks:
        xv = x_ref[:, ch * _W:(ch + 1) * _W]
        c = basec + (start + np.int32(ch * _W))
        s = _score(xv, c)
        if bv is None:
            bv, bc = s, c
        else:
            upd = s > bv
            bv = jnp.where(upd, s, bv)
            bc = jnp.where(upd, c, bc)
    return bv, bc


def _tc1_kernel(base_ref, x_ref, xt_ref, bvo_ref, bco_ref, bv_ref, bc_ref):
    p = pl.program_id(0)
    basec = base_ref[...]

    start = p * np.int32(_BLK)
    h = _NCH // 2
    bva, bca = _run_chunks(x_ref, basec, start, range(h))
    bvb, bcb = _run_chunks(x_ref, basec, start, range(h, _NCH))
    updh = bvb > bva
    bv = jnp.where(updh, bvb, bva)
    bc = jnp.where(updh, bcb, bca)

    @pl.when(p == 0)
    def _init():
        bv_ref[...] = bv
        bc_ref[...] = bc

    @pl.when(p > 0)
    def _merge():
        ov = bv_ref[...]
        upd = bv > ov
        bv_ref[...] = jnp.where(upd, bv, ov)
        bc_ref[...] = jnp.where(upd, bc, bc_ref[...])

    @pl.when(p == _NB1 - 1)
    def _fin():
        mv = bv_ref[...]
        mc = bc_ref[...]
        lane = basec - basec[:, :1]
        for tc in range(_TAIL_NCH):
            col0 = _TAIL_START + tc * _W
            xv = xt_ref[:, tc * _W:(tc + 1) * _W]
            c = basec + np.int32(col0)
            s = _score(xv, c)
            s = jnp.where(lane < np.int32(_C - col0), s, _NEG_INF)
            upd = s > mv
            mv = jnp.where(upd, s, mv)
            mc = jnp.where(upd, c, mc)
        bvo_ref[...] = mv
        bco_ref[...] = mc


def _tc2_kernel(base_ref, bvi_ref, bci_ref, x_ref, bits_ref, o_ref,
                bv_ref, bc_ref):
    p = pl.program_id(0)
    basec = base_ref[...]

    @pl.when(p == 0)
    def _init():
        bv_ref[...] = bvi_ref[...]
        bc_ref[...] = bci_ref[...]

    mv = bv_ref[...]
    mc = bc_ref[...]
    start = np.int32(_S0) + p * np.int32(_BLK)
    for ch in range(_NCH):
        bits = bits_ref[:, ch * _W:(ch + 1) * _W]
        xv = x_ref[:, ch * _W:(ch + 1) * _W]
        c = basec + (start + np.int32(ch * _W))
        s = _bits_to_score(bits, xv)
        upd = (s > mv) | ((s == mv) & (c < mc))
        mv = jnp.where(upd, s, mv)
        mc = jnp.where(upd, c, mc)
    bv_ref[...] = mv
    bc_ref[...] = mc

    @pl.when(p == _B - 1)
    def _fin():
        m = jnp.max(mv, axis=1, keepdims=True)
        arg = jnp.min(jnp.where(mv == m, mc, _IMAX), axis=1, keepdims=True)
        o_ref[...] = arg - basec[:, :1]


@jax.jit
def kernel(x):
    base = (np.arange(_R, dtype=np.int32)[:, None] * _C +
            np.arange(_W, dtype=np.int32)[None, :] + 42)
    basec = jnp.asarray(base)

    sc_bits = _sc_bits()

    bv1, bc1 = pl.pallas_call(
        _tc1_kernel,
        grid=(_NB1,),
        in_specs=[
            pl.BlockSpec((_R, _W), lambda p: (0, 0)),
            pl.BlockSpec((_R, _BLK), lambda p: (0, p)),
            pl.BlockSpec((_R, _TAIL_BLK),
                         lambda p: (0, _TAIL_START // _TAIL_BLK)),
        ],
        out_specs=[
            pl.BlockSpec((_R, _W), lambda p: (0, 0)),
            pl.BlockSpec((_R, _W), lambda p: (0, 0)),
        ],
        out_shape=[
            jax.ShapeDtypeStruct((_R, _W), jnp.float32),
            jax.ShapeDtypeStruct((_R, _W), jnp.int32),
        ],
        scratch_shapes=[
            pltpu.VMEM((_R, _W), jnp.float32),
            pltpu.VMEM((_R, _W), jnp.int32),
        ],
    )(basec, x, x)

    return pl.pallas_call(
        _tc2_kernel,
        grid=(_B,),
        in_specs=[
            pl.BlockSpec((_R, _W), lambda p: (0, 0)),
            pl.BlockSpec((_R, _W), lambda p: (0, 0)),
            pl.BlockSpec((_R, _W), lambda p: (0, 0)),
            pl.BlockSpec((_R, _BLK), lambda p: (0, _NB1 + p)),
            pl.BlockSpec((_R, _BLK), lambda p: (0, p)),
        ],
        out_specs=pl.BlockSpec((_R, 1), lambda p: (0, 0)),
        out_shape=jax.ShapeDtypeStruct((_R, 1), jnp.int32),
        scratch_shapes=[
            pltpu.VMEM((_R, _W), jnp.float32),
            pltpu.VMEM((_R, _W), jnp.int32),
        ],
    )(basec, bv1, bc1, x, sc_bits)


# speedup vs baseline: 1.1158x; 1.0523x over previous
import functools

import jax
import jax.numpy as jnp
import numpy as np
from jax import lax
from jax.experimental import pallas as pl
from jax.experimental.pallas import tpu as pltpu
from jax.experimental.pallas import tpu_sc as plsc

_R = 32
_C = 1000000
_BLK = 8192
_W = 128
_NCH = _BLK // _W
_NBF = _C // _BLK
_TAIL_START = _NBF * _BLK
_TAIL_BLK = 1024
_TAIL_NCH = _TAIL_BLK // _W

_B = 36
_NSC = _B * _BLK
_NB1 = _NBF - _B
_S0 = _NB1 * _BLK
_CH = 4096
_U = 8

_K1 = np.int32(42)
_K2 = np.int32(np.uint32(42) ^ np.uint32(0x1BD11BDA))
_ROT0 = (13, 15, 26, 6)
_ROT1 = (17, 29, 16, 24)
_TINY = np.float32(np.finfo(np.float32).tiny)
_NEG_INF = np.float32(-np.inf)
_IMAX = np.int32(2**31 - 1)


def _rotl(v, r):
    return lax.shift_left(v, np.int32(r)) | lax.shift_right_logical(
        v, np.int32(32 - r))


def _rounds(x0, x1, rots):
    for r in rots:
        x0 = x0 + x1
        x1 = _rotl(x1, r)
        x1 = x0 ^ x1
    return x0, x1


def _threefry_bits(x1):
    x0 = x1
    x1 = x0 ^ _rotl(x1, 13)
    for r in _ROT0[1:]:
        x0 = x0 + x1
        x1 = _rotl(x1, r)
        x1 = x0 ^ x1
    x0 = x0 + _K1
    x1 = x1 + np.int32(_K2 + np.uint32(1))
    x0, x1 = _rounds(x0, x1, _ROT1)
    x0 = x0 + _K2
    x1 = x1 + np.int32(2)
    x0, x1 = _rounds(x0, x1, _ROT0)
    x1 = x1 + np.int32(_K1 + np.uint32(3))
    x0, x1 = _rounds(x0, x1, _ROT1)
    x0 = x0 + _K1
    x1 = x1 + np.int32(_K2 + np.uint32(4))
    x0, x1 = _rounds(x0, x1, _ROT0)
    x0 = x0 + _K2
    x1 = x1 + np.int32(5)
    return x0 ^ x1


def _bits_to_score(bits, xv):
    fb = lax.shift_right_logical(bits, np.int32(9)) | np.int32(0x3F800000)
    f = lax.bitcast_convert_type(fb, jnp.float32) - np.float32(1.0)
    u = f + _TINY
    return -jnp.log(-jnp.log(u)) + jnp.log(xv)


def _score(xv, x1_0):
    return _bits_to_score(_threefry_bits(x1_0), xv)


def _sc_bits_body(o_hbm, va_ref, vb_ref, sa, sb):
    w = lax.axis_index("s") * 2 + lax.axis_index("c")
    base = w * np.int32(_C) + np.int32(_S0 + 42)
    lanes = lax.iota(jnp.int32, 16)

    def fill(v_ref, b0):
        def vec_body(j, c2):
            pos = j * np.int32(16 * _U)
            for k in range(_U):
                x1 = (b0 + pos + np.int32(16 * k)) + lanes
                v_ref[pl.ds(pos + np.int32(16 * k), 16)] = _threefry_bits(x1)
            return c2

        lax.fori_loop(0, _CH // (16 * _U), vec_body, 0)

    def pair_body(ci, carry):
        off = ci * np.int32(2 * _CH)

        @pl.when(ci > 0)
        def _wait_a():
            pltpu.make_async_copy(va_ref, o_hbm.at[w, pl.ds(0, _CH)],
                                  sa).wait()

        fill(va_ref, base + off)
        pltpu.async_copy(va_ref, o_hbm.at[w, pl.ds(off, _CH)], sa)

        @pl.when(ci > 0)
        def _wait_b():
            pltpu.make_async_copy(vb_ref, o_hbm.at[w, pl.ds(0, _CH)],
                                  sb).wait()

        fill(vb_ref, base + off + np.int32(_CH))
        pltpu.async_copy(
            vb_ref, o_hbm.at[w, pl.ds(off + np.int32(_CH), _CH)], sb)
        return carry

    lax.fori_loop(0, _NSC // (2 * _CH), pair_body, 0)
    pltpu.make_async_copy(va_ref, o_hbm.at[w, pl.ds(0, _CH)], sa).wait()
    pltpu.make_async_copy(vb_ref, o_hbm.at[w, pl.ds(0, _CH)], sb).wait()


def _sc_bits():
    mesh = plsc.VectorSubcoreMesh(core_axis_name="c", subcore_axis_name="s")
    return pl.kernel(
        _sc_bits_body,
        mesh=mesh,
        out_type=jax.ShapeDtypeStruct((_R, _NSC), jnp.int32),
        scratch_types=[pltpu.VMEM((_CH,), jnp.int32),
                       pltpu.VMEM((_CH,), jnp.int32),
                       pltpu.SemaphoreType.DMA,
                       pltpu.SemaphoreType.DMA],
    )()


def _run_chunks(x_ref, basec, start, chunks):
    bv = bc = None
    for ch in chunks:
        xv = x_ref[:, ch * _W:(ch + 1) * _W]
        c = basec + (start + np.int32(ch * _W))
        s = _score(xv, c)
        if bv is None:
            bv, bc = s, c
        else:
            upd = s > bv
            bv = jnp.where(upd, s, bv)
            bc = jnp.where(upd, c, bc)
    return bv, bc


def _tc1_kernel(base_ref, x_ref, xt_ref, bvo_ref, bco_ref, bv_ref, bc_ref):
    p = pl.program_id(0)
    basec = base_ref[...]

    start = p * np.int32(_BLK)
    h = _NCH // 2
    bva, bca = _run_chunks(x_ref, basec, start, range(h))
    bvb, bcb = _run_chunks(x_ref, basec, start, range(h, _NCH))
    updh = bvb > bva
    bv = jnp.where(updh, bvb, bva)
    bc = jnp.where(updh, bcb, bca)

    @pl.when(p == 0)
    def _init():
        bv_ref[...] = bv
        bc_ref[...] = bc

    @pl.when(p > 0)
    def _merge():
        ov = bv_ref[...]
        upd = bv > ov
        bv_ref[...] = jnp.where(upd, bv, ov)
        bc_ref[...] = jnp.where(upd, bc, bc_ref[...])

    @pl.when(p == _NB1 - 1)
    def _fin():
        mv = bv_ref[...]
        mc = bc_ref[...]
        lane = basec - basec[:, :1]
        for tc in range(_TAIL_NCH):
            col0 = _TAIL_START + tc * _W
            xv = xt_ref[:, tc * _W:(tc + 1) * _W]
            c = basec + np.int32(col0)
            s = _score(xv, c)
            s = jnp.where(lane < np.int32(_C - col0), s, _NEG_INF)
            upd = s > mv
            mv = jnp.where(upd, s, mv)
            mc = jnp.where(upd, c, mc)
        bvo_ref[...] = mv
        bco_ref[...] = mc


def _tc2_kernel(base_ref, bvi_ref, bci_ref, x_ref, bits_ref, o_ref,
                bv_ref, bc_ref):
    p = pl.program_id(0)
    basec = base_ref[...]

    @pl.when(p == 0)
    def _init():
        bv_ref[...] = bvi_ref[...]
        bc_ref[...] = bci_ref[...]

    mv = bv_ref[...]
    mc = bc_ref[...]
    start = np.int32(_S0) + p * np.int32(_BLK)
    for ch in range(_NCH):
        bits = bits_ref[:, ch * _W:(ch + 1) * _W]
        xv = x_ref[:, ch * _W:(ch + 1) * _W]
        c = basec + (start + np.int32(ch * _W))
        s = _bits_to_score(bits, xv)
        upd = (s > mv) | ((s == mv) & (c < mc))
        mv = jnp.where(upd, s, mv)
        mc = jnp.where(upd, c, mc)
    bv_ref[...] = mv
    bc_ref[...] = mc

    @pl.when(p == _B - 1)
    def _fin():
        m = jnp.max(mv, axis=1, keepdims=True)
        arg = jnp.min(jnp.where(mv == m, mc, _IMAX), axis=1, keepdims=True)
        o_ref[...] = arg - basec[:, :1]


@jax.jit
def kernel(x):
    base = (np.arange(_R, dtype=np.int32)[:, None] * _C +
            np.arange(_W, dtype=np.int32)[None, :] + 42)
    basec = jnp.asarray(base)

    sc_bits = _sc_bits()

    bv1, bc1 = pl.pallas_call(
        _tc1_kernel,
        grid=(_NB1,),
        in_specs=[
            pl.BlockSpec((_R, _W), lambda p: (0, 0)),
            pl.BlockSpec((_R, _BLK), lambda p: (0, p)),
            pl.BlockSpec((_R, _TAIL_BLK),
                         lambda p: (0, _TAIL_START // _TAIL_BLK)),
        ],
        out_specs=[
            pl.BlockSpec((_R, _W), lambda p: (0, 0)),
            pl.BlockSpec((_R, _W), lambda p: (0, 0)),
        ],
        out_shape=[
            jax.ShapeDtypeStruct((_R, _W), jnp.float32),
            jax.ShapeDtypeStruct((_R, _W), jnp.int32),
        ],
        scratch_shapes=[
            pltpu.VMEM((_R, _W), jnp.float32),
            pltpu.VMEM((_R, _W), jnp.int32),
        ],
    )(basec, x, x)

    return pl.pallas_call(
        _tc2_kernel,
        grid=(_B,),
        in_specs=[
            pl.BlockSpec((_R, _W), lambda p: (0, 0)),
            pl.BlockSpec((_R, _W), lambda p: (0, 0)),
            pl.BlockSpec((_R, _W), lambda p: (0, 0)),
            pl.BlockSpec((_R, _BLK), lambda p: (0, _NB1 + p)),
            pl.BlockSpec((_R, _BLK), lambda p: (0, p)),
        ],
        out_specs=pl.BlockSpec((_R, 1), lambda p: (0, 0)),
        out_shape=jax.ShapeDtypeStruct((_R, 1), jnp.int32),
        scratch_shapes=[
            pltpu.VMEM((_R, _W), jnp.float32),
            pltpu.VMEM((_R, _W), jnp.int32),
        ],
    )(basec, bv1, bc1, x, sc_bits)


# speedup vs baseline: 1.1720x; 1.0504x over previous
import functools

import jax
import jax.numpy as jnp
import numpy as np
from jax import lax
from jax.experimental import pallas as pl
from jax.experimental.pallas import tpu as pltpu
from jax.experimental.pallas import tpu_sc as plsc

_R = 32
_C = 1000000
_BLK = 8192
_W = 128
_NCH = _BLK // _W
_NBF = _C // _BLK
_TAIL_START = _NBF * _BLK
_TAIL_BLK = 1024
_TAIL_NCH = _TAIL_BLK // _W

_B = 33
_NSC = _B * _BLK
_NB1 = _NBF - _B
_S0 = _NB1 * _BLK
_CH = 4096
_U = 8

_K1 = np.int32(42)
_K2 = np.int32(np.uint32(42) ^ np.uint32(0x1BD11BDA))
_ROT0 = (13, 15, 26, 6)
_ROT1 = (17, 29, 16, 24)
_TINY = np.float32(np.finfo(np.float32).tiny)
_NEG_INF = np.float32(-np.inf)
_IMAX = np.int32(2**31 - 1)


def _rotl(v, r):
    return lax.shift_left(v, np.int32(r)) | lax.shift_right_logical(
        v, np.int32(32 - r))


def _rounds(x0, x1, rots):
    for r in rots:
        x0 = x0 + x1
        x1 = _rotl(x1, r)
        x1 = x0 ^ x1
    return x0, x1


def _threefry_bits(x1):
    x0 = x1
    x1 = x0 ^ _rotl(x1, 13)
    for r in _ROT0[1:]:
        x0 = x0 + x1
        x1 = _rotl(x1, r)
        x1 = x0 ^ x1
    x0 = x0 + _K1
    x1 = x1 + np.int32(_K2 + np.uint32(1))
    x0, x1 = _rounds(x0, x1, _ROT1)
    x0 = x0 + _K2
    x1 = x1 + np.int32(2)
    x0, x1 = _rounds(x0, x1, _ROT0)
    x1 = x1 + np.int32(_K1 + np.uint32(3))
    x0, x1 = _rounds(x0, x1, _ROT1)
    x0 = x0 + _K1
    x1 = x1 + np.int32(_K2 + np.uint32(4))
    x0, x1 = _rounds(x0, x1, _ROT0)
    x0 = x0 + _K2
    x1 = x1 + np.int32(5)
    return x0 ^ x1


def _bits_to_score(bits, xv):
    fb = lax.shift_right_logical(bits, np.int32(9)) | np.int32(0x3F800000)
    f = lax.bitcast_convert_type(fb, jnp.float32) - np.float32(1.0)
    u = f + _TINY
    return -jnp.log(-jnp.log(u)) + jnp.log(xv)


def _score(xv, x1_0):
    return _bits_to_score(_threefry_bits(x1_0), xv)


def _sc_bits_body(o_hbm, va_ref, vb_ref, sa, sb):
    w = lax.axis_index("s") * 2 + lax.axis_index("c")
    base = w * np.int32(_C) + np.int32(_S0 + 42)
    lanes = lax.iota(jnp.int32, 16)

    def fill(v_ref, b0):
        def vec_body(j, c2):
            pos = j * np.int32(16 * _U)
            for k in range(_U):
                x1 = (b0 + pos + np.int32(16 * k)) + lanes
                v_ref[pl.ds(pos + np.int32(16 * k), 16)] = _threefry_bits(x1)
            return c2

        lax.fori_loop(0, _CH // (16 * _U), vec_body, 0)

    def pair_body(ci, carry):
        off = ci * np.int32(2 * _CH)

        @pl.when(ci > 0)
        def _wait_a():
            pltpu.make_async_copy(va_ref, o_hbm.at[w, pl.ds(0, _CH)],
                                  sa).wait()

        fill(va_ref, base + off)
        pltpu.async_copy(va_ref, o_hbm.at[w, pl.ds(off, _CH)], sa)

        @pl.when(ci > 0)
        def _wait_b():
            pltpu.make_async_copy(vb_ref, o_hbm.at[w, pl.ds(0, _CH)],
                                  sb).wait()

        fill(vb_ref, base + off + np.int32(_CH))
        pltpu.async_copy(
            vb_ref, o_hbm.at[w, pl.ds(off + np.int32(_CH), _CH)], sb)
        return carry

    lax.fori_loop(0, _NSC // (2 * _CH), pair_body, 0)
    pltpu.make_async_copy(va_ref, o_hbm.at[w, pl.ds(0, _CH)], sa).wait()
    pltpu.make_async_copy(vb_ref, o_hbm.at[w, pl.ds(0, _CH)], sb).wait()


def _sc_bits():
    mesh = plsc.VectorSubcoreMesh(core_axis_name="c", subcore_axis_name="s")
    return pl.kernel(
        _sc_bits_body,
        mesh=mesh,
        out_type=jax.ShapeDtypeStruct((_R, _NSC), jnp.int32),
        scratch_types=[pltpu.VMEM((_CH,), jnp.int32),
                       pltpu.VMEM((_CH,), jnp.int32),
                       pltpu.SemaphoreType.DMA,
                       pltpu.SemaphoreType.DMA],
    )()


def _run_chunks(x_ref, basec, start, chunks):
    bv = bc = None
    for ch in chunks:
        xv = x_ref[:, ch * _W:(ch + 1) * _W]
        c = basec + (start + np.int32(ch * _W))
        s = _score(xv, c)
        if bv is None:
            bv, bc = s, c
        else:
            upd = s > bv
            bv = jnp.where(upd, s, bv)
            bc = jnp.where(upd, c, bc)
    return bv, bc


def _tc1_kernel(base_ref, x_ref, xt_ref, bvo_ref, bco_ref, bv_ref, bc_ref):
    p = pl.program_id(0)
    basec = base_ref[...]

    start = p * np.int32(_BLK)
    h = _NCH // 2
    bva, bca = _run_chunks(x_ref, basec, start, range(h))
    bvb, bcb = _run_chunks(x_ref, basec, start, range(h, _NCH))
    updh = bvb > bva
    bv = jnp.where(updh, bvb, bva)
    bc = jnp.where(updh, bcb, bca)

    @pl.when(p == 0)
    def _init():
        bv_ref[...] = bv
        bc_ref[...] = bc

    @pl.when(p > 0)
    def _merge():
        ov = bv_ref[...]
        upd = bv > ov
        bv_ref[...] = jnp.where(upd, bv, ov)
        bc_ref[...] = jnp.where(upd, bc, bc_ref[...])

    @pl.when(p == _NB1 - 1)
    def _fin():
        mv = bv_ref[...]
        mc = bc_ref[...]
        lane = basec - basec[:, :1]
        for tc in range(_TAIL_NCH):
            col0 = _TAIL_START + tc * _W
            xv = xt_ref[:, tc * _W:(tc + 1) * _W]
            c = basec + np.int32(col0)
            s = _score(xv, c)
            s = jnp.where(lane < np.int32(_C - col0), s, _NEG_INF)
            upd = s > mv
            mv = jnp.where(upd, s, mv)
            mc = jnp.where(upd, c, mc)
        bvo_ref[...] = mv
        bco_ref[...] = mc


def _tc2_kernel(base_ref, bvi_ref, bci_ref, x_ref, bits_ref, o_ref,
                bv_ref, bc_ref):
    p = pl.program_id(0)
    basec = base_ref[...]

    @pl.when(p == 0)
    def _init():
        bv_ref[...] = bvi_ref[...]
        bc_ref[...] = bci_ref[...]

    mv = bv_ref[...]
    mc = bc_ref[...]
    start = np.int32(_S0) + p * np.int32(_BLK)
    for ch in range(_NCH):
        bits = bits_ref[:, ch * _W:(ch + 1) * _W]
        xv = x_ref[:, ch * _W:(ch + 1) * _W]
        c = basec + (start + np.int32(ch * _W))
        s = _bits_to_score(bits, xv)
        upd = (s > mv) | ((s == mv) & (c < mc))
        mv = jnp.where(upd, s, mv)
        mc = jnp.where(upd, c, mc)
    bv_ref[...] = mv
    bc_ref[...] = mc

    @pl.when(p == _B - 1)
    def _fin():
        m = jnp.max(mv, axis=1, keepdims=True)
        arg = jnp.min(jnp.where(mv == m, mc, _IMAX), axis=1, keepdims=True)
        o_ref[...] = arg - basec[:, :1]


@jax.jit
def kernel(x):
    base = (np.arange(_R, dtype=np.int32)[:, None] * _C +
            np.arange(_W, dtype=np.int32)[None, :] + 42)
    basec = jnp.asarray(base)

    sc_bits = _sc_bits()

    bv1, bc1 = pl.pallas_call(
        _tc1_kernel,
        grid=(_NB1,),
        in_specs=[
            pl.BlockSpec((_R, _W), lambda p: (0, 0)),
            pl.BlockSpec((_R, _BLK), lambda p: (0, p)),
            pl.BlockSpec((_R, _TAIL_BLK),
                         lambda p: (0, _TAIL_START // _TAIL_BLK)),
        ],
        out_specs=[
            pl.BlockSpec((_R, _W), lambda p: (0, 0)),
            pl.BlockSpec((_R, _W), lambda p: (0, 0)),
        ],
        out_shape=[
            jax.ShapeDtypeStruct((_R, _W), jnp.float32),
            jax.ShapeDtypeStruct((_R, _W), jnp.int32),
        ],
        scratch_shapes=[
            pltpu.VMEM((_R, _W), jnp.float32),
            pltpu.VMEM((_R, _W), jnp.int32),
        ],
    )(basec, x, x)

    return pl.pallas_call(
        _tc2_kernel,
        grid=(_B,),
        in_specs=[
            pl.BlockSpec((_R, _W), lambda p: (0, 0)),
            pl.BlockSpec((_R, _W), lambda p: (0, 0)),
            pl.BlockSpec((_R, _W), lambda p: (0, 0)),
            pl.BlockSpec((_R, _BLK), lambda p: (0, _NB1 + p)),
            pl.BlockSpec((_R, _BLK), lambda p: (0, p)),
        ],
        out_specs=pl.BlockSpec((_R, 1), lambda p: (0, 0)),
        out_shape=jax.ShapeDtypeStruct((_R, 1), jnp.int32),
        scratch_shapes=[
            pltpu.VMEM((_R, _W), jnp.float32),
            pltpu.VMEM((_R, _W), jnp.int32),
        ],
    )(basec, bv1, bc1, x, sc_bits)


# speedup vs baseline: 1.1762x; 1.0036x over previous
import jax
import jax.numpy as jnp
import numpy as np
from jax import lax
from jax.experimental import pallas as pl
from jax.experimental.pallas import tpu as pltpu
from jax.experimental.pallas import tpu_sc as plsc

_R = 32
_C = 1000000
_BLK = 8192
_W = 128
_NCH = _BLK // _W
_NBF = _C // _BLK
_TAIL_START = _NBF * _BLK
_TAIL_BLK = 1024
_TAIL_NCH = _TAIL_BLK // _W

_B = 34
_NSC = _B * _BLK
_NB1 = _NBF - _B
_S0 = _NB1 * _BLK
_CH = 4096
_U = 8

_K1 = np.int32(42)
_K2 = np.int32(np.uint32(42) ^ np.uint32(0x1BD11BDA))
_ROT0 = (13, 15, 26, 6)
_ROT1 = (17, 29, 16, 24)
_TINY = np.float32(np.finfo(np.float32).tiny)
_NEG_INF = np.float32(-np.inf)
_IMAX = np.int32(2**31 - 1)


def _rotl(v, r):
    return lax.shift_left(v, np.int32(r)) | lax.shift_right_logical(
        v, np.int32(32 - r))


def _rounds(x0, x1, rots):
    for r in rots:
        x0 = x0 + x1
        x1 = _rotl(x1, r)
        x1 = x0 ^ x1
    return x0, x1


def _threefry_bits(x1):
    x0 = x1
    x1 = x0 ^ _rotl(x1, 13)
    for r in _ROT0[1:]:
        x0 = x0 + x1
        x1 = _rotl(x1, r)
        x1 = x0 ^ x1
    x0 = x0 + _K1
    x1 = x1 + np.int32(_K2 + np.uint32(1))
    x0, x1 = _rounds(x0, x1, _ROT1)
    x0 = x0 + _K2
    x1 = x1 + np.int32(2)
    x0, x1 = _rounds(x0, x1, _ROT0)
    x1 = x1 + np.int32(_K1 + np.uint32(3))
    x0, x1 = _rounds(x0, x1, _ROT1)
    x0 = x0 + _K1
    x1 = x1 + np.int32(_K2 + np.uint32(4))
    x0, x1 = _rounds(x0, x1, _ROT0)
    x0 = x0 + _K2
    x1 = x1 + np.int32(5)
    return x0 ^ x1


def _bits_to_score(bits, xv):
    fb = lax.shift_right_logical(bits, np.int32(9)) | np.int32(0x3F800000)
    f = lax.bitcast_convert_type(fb, jnp.float32) - np.float32(1.0)
    u = f + _TINY
    return -jnp.log(-jnp.log(u)) + jnp.log(xv)


def _score(xv, x1_0):
    return _bits_to_score(_threefry_bits(x1_0), xv)


def _sc_bits_body(o_hbm, va_ref, vb_ref, sa, sb):
    w = lax.axis_index("s") * 2 + lax.axis_index("c")
    base = w * np.int32(_C) + np.int32(_S0 + 42)
    lanes = lax.iota(jnp.int32, 16)

    def fill(v_ref, b0):
        def vec_body(j, c2):
            pos = j * np.int32(16 * _U)
            for k in range(_U):
                x1 = (b0 + pos + np.int32(16 * k)) + lanes
                v_ref[pl.ds(pos + np.int32(16 * k), 16)] = _threefry_bits(x1)
            return c2

        lax.fori_loop(0, _CH // (16 * _U), vec_body, 0)

    def pair_body(ci, carry):
        off = ci * np.int32(2 * _CH)

        @pl.when(ci > 0)
        def _wait_a():
            pltpu.make_async_copy(va_ref, o_hbm.at[w, pl.ds(0, _CH)],
                                  sa).wait()

        fill(va_ref, base + off)
        pltpu.async_copy(va_ref, o_hbm.at[w, pl.ds(off, _CH)], sa)

        @pl.when(ci > 0)
        def _wait_b():
            pltpu.make_async_copy(vb_ref, o_hbm.at[w, pl.ds(0, _CH)],
                                  sb).wait()

        fill(vb_ref, base + off + np.int32(_CH))
        pltpu.async_copy(
            vb_ref, o_hbm.at[w, pl.ds(off + np.int32(_CH), _CH)], sb)
        return carry

    lax.fori_loop(0, _NSC // (2 * _CH), pair_body, 0)
    pltpu.make_async_copy(va_ref, o_hbm.at[w, pl.ds(0, _CH)], sa).wait()
    pltpu.make_async_copy(vb_ref, o_hbm.at[w, pl.ds(0, _CH)], sb).wait()


def _sc_bits():
    mesh = plsc.VectorSubcoreMesh(core_axis_name="c", subcore_axis_name="s")
    return pl.kernel(
        _sc_bits_body,
        mesh=mesh,
        out_type=jax.ShapeDtypeStruct((_R, _NSC), jnp.int32),
        scratch_types=[pltpu.VMEM((_CH,), jnp.int32),
                       pltpu.VMEM((_CH,), jnp.int32),
                       pltpu.SemaphoreType.DMA,
                       pltpu.SemaphoreType.DMA],
    )()


def _run_chunks(x_ref, basec, start, chunks):
    bv = bc = None
    for ch in chunks:
        xv = x_ref[:, ch * _W:(ch + 1) * _W]
        c = basec + (start + np.int32(ch * _W))
        s = _score(xv, c)
        if bv is None:
            bv, bc = s, c
        else:
            upd = s > bv
            bv = jnp.where(upd, s, bv)
            bc = jnp.where(upd, c, bc)
    return bv, bc


def _tc1_kernel(base_ref, x_ref, xt_ref, bvo_ref, bco_ref, bv_ref, bc_ref):
    p = pl.program_id(0)
    basec = base_ref[...]

    start = p * np.int32(_BLK)
    h = _NCH // 2
    bva, bca = _run_chunks(x_ref, basec, start, range(h))
    bvb, bcb = _run_chunks(x_ref, basec, start, range(h, _NCH))
    updh = bvb > bva
    bv = jnp.where(updh, bvb, bva)
    bc = jnp.where(updh, bcb, bca)

    @pl.when(p == 0)
    def _init():
        bv_ref[...] = bv
        bc_ref[...] = bc

    @pl.when(p > 0)
    def _merge():
        ov = bv_ref[...]
        upd = bv > ov
        bv_ref[...] = jnp.where(upd, bv, ov)
        bc_ref[...] = jnp.where(upd, bc, bc_ref[...])

    @pl.when(p == _NB1 - 1)
    def _fin():
        mv = bv_ref[...]
        mc = bc_ref[...]
        lane = basec - basec[:, :1]
        for tc in range(_TAIL_NCH):
            col0 = _TAIL_START + tc * _W
            xv = xt_ref[:, tc * _W:(tc + 1) * _W]
            c = basec + np.int32(col0)
            s = _score(xv, c)
            s = jnp.where(lane < np.int32(_C - col0), s, _NEG_INF)
            upd = s > mv
            mv = jnp.where(upd, s, mv)
            mc = jnp.where(upd, c, mc)
        bvo_ref[...] = mv
        bco_ref[...] = mc


def _tc2_kernel(base_ref, bvi_ref, bci_ref, x_ref, bits_ref, o_ref,
                bv_ref, bc_ref):
    p = pl.program_id(0)
    basec = base_ref[...]

    @pl.when(p == 0)
    def _init():
        bv_ref[...] = bvi_ref[...]
        bc_ref[...] = bci_ref[...]

    mv = bv_ref[...]
    mc = bc_ref[...]
    start = np.int32(_S0) + p * np.int32(_BLK)
    for ch in range(_NCH):
        bits = bits_ref[:, ch * _W:(ch + 1) * _W]
        xv = x_ref[:, ch * _W:(ch + 1) * _W]
        c = basec + (start + np.int32(ch * _W))
        s = _bits_to_score(bits, xv)
        upd = (s > mv) | ((s == mv) & (c < mc))
        mv = jnp.where(upd, s, mv)
        mc = jnp.where(upd, c, mc)
    bv_ref[...] = mv
    bc_ref[...] = mc

    @pl.when(p == _B - 1)
    def _fin():
        m = jnp.max(mv, axis=1, keepdims=True)
        arg = jnp.min(jnp.where(mv == m, mc, _IMAX), axis=1, keepdims=True)
        o_ref[...] = arg - basec[:, :1]


@jax.jit
def kernel(x):
    base = (np.arange(_R, dtype=np.int32)[:, None] * _C +
            np.arange(_W, dtype=np.int32)[None, :] + 42)
    basec = jnp.asarray(base)

    sc_bits = _sc_bits()

    bv1, bc1 = pl.pallas_call(
        _tc1_kernel,
        grid=(_NB1,),
        in_specs=[
            pl.BlockSpec((_R, _W), lambda p: (0, 0)),
            pl.BlockSpec((_R, _BLK), lambda p: (0, p)),
            pl.BlockSpec((_R, _TAIL_BLK),
                         lambda p: (0, _TAIL_START // _TAIL_BLK)),
        ],
        out_specs=[
            pl.BlockSpec((_R, _W), lambda p: (0, 0)),
            pl.BlockSpec((_R, _W), lambda p: (0, 0)),
        ],
        out_shape=[
            jax.ShapeDtypeStruct((_R, _W), jnp.float32),
            jax.ShapeDtypeStruct((_R, _W), jnp.int32),
        ],
        scratch_shapes=[
            pltpu.VMEM((_R, _W), jnp.float32),
            pltpu.VMEM((_R, _W), jnp.int32),
        ],
    )(basec, x, x)

    return pl.pallas_call(
        _tc2_kernel,
        grid=(_B,),
        in_specs=[
            pl.BlockSpec((_R, _W), lambda p: (0, 0)),
            pl.BlockSpec((_R, _W), lambda p: (0, 0)),
            pl.BlockSpec((_R, _W), lambda p: (0, 0)),
            pl.BlockSpec((_R, _BLK), lambda p: (0, _NB1 + p)),
            pl.BlockSpec((_R, _BLK), lambda p: (0, p)),
        ],
        out_specs=pl.BlockSpec((_R, 1), lambda p: (0, 0)),
        out_shape=jax.ShapeDtypeStruct((_R, 1), jnp.int32),
        scratch_shapes=[
            pltpu.VMEM((_R, _W), jnp.float32),
            pltpu.VMEM((_R, _W), jnp.int32),
        ],
    )(basec, bv1, bc1, x, sc_bits)
